# baseline (speedup 1.0000x reference)
import functools

import jax
import jax.numpy as jnp
from jax import lax
from jax.experimental import pallas as pl
from jax.experimental.pallas import tpu as pltpu
from jax.experimental.pallas import tpu_sc as plsc


def _tc_repack(table_t, BLK=1024):
    C, V = table_t.shape

    def body(x_ref, o_ref):
        x = x_ref[...]
        y = x.T.reshape(BLK // 2, 2, C)
        o_ref[...] = jnp.concatenate([y[:, 0, :], y[:, 1, :]], axis=-1)

    grid = (V + BLK - 1) // BLK
    return pl.pallas_call(
        body,
        grid=(grid,),
        in_specs=[pl.BlockSpec((C, BLK), lambda i: (0, i))],
        out_specs=pl.BlockSpec((BLK // 2, 2 * C), lambda i: (i, 0)),
        out_shape=jax.ShapeDtypeStruct((V // 2, 2 * C), jnp.float32),
        compiler_params=pltpu.CompilerParams(
            dimension_semantics=("arbitrary",)
        ),
    )(table_t)


def _make_sc_gather(B, G, D2):
    info = plsc.get_sparse_core_info()
    NC, NS, L = info.num_cores, info.num_subcores, info.num_lanes
    NW = NC * NS
    assert B % (8 * NW) == 0
    b_per_w = B // NW
    mesh = plsc.VectorSubcoreMesh(core_axis_name="c", subcore_axis_name="s")
    D = D2 // 2

    @functools.partial(
        pl.kernel,
        mesh=mesh,
        out_type=jax.ShapeDtypeStruct((B, D2), jnp.float32),
        scratch_types=[
            pltpu.VMEM((b_per_w,), jnp.int32),
            pltpu.VMEM((b_per_w,), jnp.int32),
            pltpu.VMEM((b_per_w, D2), jnp.float32),
            pltpu.SemaphoreType.DMA,
        ],
        compiler_params=pltpu.CompilerParams(needs_layout_passes=False),
    )
    def gather(ids_hbm, lin_hbm, out_hbm, idx_v, pair_v, rows_v, sem):
        wid = lax.axis_index("s") * NC + lax.axis_index("c")
        base = wid * b_per_w
        pltpu.sync_copy(ids_hbm.at[pl.ds(base, b_per_w)], idx_v)

        def compute_pairs(i, carry):
            v = idx_v[pl.ds(i * L, L)]
            pair_v[pl.ds(i * L, L)] = lax.shift_right_logical(v, 1)
            return carry

        lax.fori_loop(0, b_per_w // L, compute_pairs, 0)
        pltpu.async_copy(lin_hbm.at[pair_v], rows_v, sem).wait()

        def extract(g, carry):
            v = idx_v[pl.ds(g * L, L)]
            j = (v & 1) * D
            ivec = lax.iota(jnp.int32, L) + g * L
            for col in range(D):
                cvec = jnp.full((L,), col, jnp.int32)
                x = plsc.load_gather(rows_v, [ivec, j + cvec])
                plsc.store_scatter(rows_v, [ivec, cvec], x)
            return carry

        lax.fori_loop(0, b_per_w // L, extract, 0)
        pltpu.sync_copy(rows_v, out_hbm.at[pl.ds(base, b_per_w)])

    return gather


def kernel(item_ids, emb_weight):
    B, = item_ids.shape
    V, D = emb_weight.shape
    ids = item_ids.astype(jnp.int32)
    lin = _tc_repack(emb_weight.T)
    wide = _make_sc_gather(B, V // 2, 2 * D)(ids, lin)
    return wide[:, :D]
